# baseline (speedup 1.0000x reference)
import functools

import jax
import jax.numpy as jnp
from jax import lax
from jax.experimental import pallas as pl
from jax.experimental.pallas import tpu as pltpu
from jax.experimental.pallas import tpu_sc as plsc

VOCAB = 1000000
D = 32
B = 4096
L = 200
LPAD = 208
NC = 2
NS = 16
NW = NC * NS
BPW = B // NW
CH0, CH1 = 104, 96
NBUF = 4

_mesh = plsc.VectorSubcoreMesh(core_axis_name="c", subcore_axis_name="s")


@functools.partial(
    pl.kernel,
    mesh=_mesh,
    out_type=jax.ShapeDtypeStruct((B * D,), jnp.float32),
    scratch_types=[
        pltpu.VMEM((LPAD, BPW), jnp.int32),
        [pltpu.VMEM((LPAD,), jnp.int32) for _ in range(NBUF)],
        [pltpu.VMEM((L, D), jnp.float32) for _ in range(NBUF)],
        pltpu.VMEM((BPW * D,), jnp.float32),
        [pltpu.SemaphoreType.DMA for _ in range(NBUF)],
    ],
    compiler_params=pltpu.CompilerParams(
        use_tc_tiling_on_sc=False, needs_layout_passes=False),
)
def _emb_avg(table_hbm, idxt_hbm, out_hbm, idx_v, stages, bufs, out_v, sems):
    wid = lax.axis_index("s") * NC + lax.axis_index("c")
    pltpu.sync_copy(idxt_hbm.at[:, pl.ds(wid * BPW, BPW)], idx_v.at[pl.ds(0, L)])

    def fire(row, stage, buf, sem):
        for t in range(LPAD // 16):
            rows = lax.iota(jnp.int32, 16) + (16 * t)
            cols = jnp.full((16,), row, jnp.int32)
            stage[pl.ds(16 * t, 16)] = plsc.load_gather(idx_v, [rows, cols])
        pltpu.async_copy(
            table_hbm.at[stage.at[pl.ds(0, CH0)]],
            buf.at[pl.ds(0, CH0)], sem)
        pltpu.async_copy(
            table_hbm.at[stage.at[pl.ds(CH0, CH1)]],
            buf.at[pl.ds(CH0, CH1)], sem)

    def drain(buf, sem):
        pltpu.make_async_copy(
            table_hbm.at[pl.ds(0, CH0)], buf.at[pl.ds(0, CH0)], sem).wait()
        pltpu.make_async_copy(
            table_hbm.at[pl.ds(0, CH1)], buf.at[pl.ds(CH0, CH1)], sem).wait()

    def reduce_store(row, buf):
        accs = [jnp.zeros((16,), jnp.float32) for _ in range(8)]
        for j in range(L):
            k = (j % 4) * 2
            accs[k] = accs[k] + buf[j, 0:16]
            accs[k + 1] = accs[k + 1] + buf[j, 16:32]
        r0 = ((accs[0] + accs[2]) + (accs[4] + accs[6])) * (1.0 / L)
        r1 = ((accs[1] + accs[3]) + (accs[5] + accs[7])) * (1.0 / L)
        out_v[pl.ds(row * D, 16)] = r0
        out_v[pl.ds(row * D + 16, 16)] = r1

    for s in range(NBUF):
        fire(s, stages[s], bufs[s], sems[s])

    def body(g, carry):
        for s in range(NBUF):
            row = g * NBUF + s
            drain(bufs[s], sems[s])
            reduce_store(row, bufs[s])

            @pl.when(row + NBUF < BPW)
            def _():
                fire(row + NBUF, stages[s], bufs[s], sems[s])

        return carry

    lax.fori_loop(0, BPW // NBUF, body, 0)
    pltpu.sync_copy(out_v, out_hbm.at[pl.ds(wid * (BPW * D), BPW * D)])


def kernel(embedding_table, input):
    out = _emb_avg(embedding_table, input.T)
    return out.reshape(B, 1, D)
